# baseline (speedup 1.0000x reference)
import functools

import jax
import jax.numpy as jnp
from jax import lax
from jax.experimental import pallas as pl
from jax.experimental.pallas import tpu as pltpu
from jax.experimental.pallas import tpu_sc as plsc

B = 131072
D = 64
K = 512

BLOCK_B = 2048
GRID = B // BLOCK_B

NUM_WORKERS = 32
BPW = B // NUM_WORKERS
CHUNK = 1024
NCHUNK = BPW // CHUNK


def _argmin_body(x_ref, cbt_ref, idx_ref, loss_ref):
    x = x_ref[...]
    cbt = cbt_ref[...]
    dot = lax.dot_general(x, cbt, (((1,), (0,)), ((), ())),
                          preferred_element_type=jnp.float32)
    xnorm = jnp.sum(x * x, axis=1, keepdims=True)
    cnorm = jnp.sum(cbt * cbt, axis=0, keepdims=True)
    dist = (xnorm + cnorm) - 2.0 * dot
    minval = jnp.min(dist, axis=1, keepdims=True)
    iota = lax.broadcasted_iota(jnp.int32, dist.shape, 1)
    idx = jnp.min(jnp.where(dist == minval, iota, K), axis=1, keepdims=True)
    idx_ref[...] = idx

    @pl.when(pl.program_id(0) == 0)
    def _():
        loss_ref[...] = jnp.zeros_like(loss_ref)

    loss_ref[...] += jnp.sum(minval)


def _tc_argmin(inputs, cbt):
    return pl.pallas_call(
        _argmin_body,
        grid=(GRID,),
        in_specs=[
            pl.BlockSpec((BLOCK_B, D), lambda i: (i, 0)),
            pl.BlockSpec((D, K), lambda i: (0, 0)),
        ],
        out_specs=[
            pl.BlockSpec((BLOCK_B, 1), lambda i: (i, 0)),
            pl.BlockSpec((1, 1), lambda i: (0, 0)),
        ],
        out_shape=[
            jax.ShapeDtypeStruct((B, 1), jnp.int32),
            jax.ShapeDtypeStruct((1, 1), jnp.float32),
        ],
    )(inputs, cbt)


def _sc_gather_body(table_hbm, idx_hbm, out_hbm, idx_v, rows_v, sem):
    info = plsc.get_sparse_core_info()
    wid = lax.axis_index("s") * info.num_cores + lax.axis_index("c")
    base = wid * BPW
    pltpu.sync_copy(idx_hbm.at[pl.ds(base, BPW)], idx_v)
    for c in range(NCHUNK):
        pltpu.async_copy(
            table_hbm.at[idx_v.at[pl.ds(c * CHUNK, CHUNK)]], rows_v, sem
        ).wait()
        pltpu.sync_copy(rows_v, out_hbm.at[pl.ds(base + c * CHUNK, CHUNK)])


@functools.lru_cache(maxsize=1)
def _make_sc_gather():
    return pl.kernel(
        _sc_gather_body,
        mesh=plsc.VectorSubcoreMesh(core_axis_name="c", subcore_axis_name="s"),
        compiler_params=pltpu.CompilerParams(use_tc_tiling_on_sc=False),
        out_type=jax.ShapeDtypeStruct((B, D), jnp.float32),
        scratch_types=[
            pltpu.VMEM((BPW,), jnp.int32),
            pltpu.VMEM((CHUNK, D), jnp.float32),
            pltpu.SemaphoreType.DMA,
        ],
    )


def kernel(inputs, codebook):
    cbt = codebook.T
    idx2d, loss_sum = _tc_argmin(inputs, cbt)
    encoding_indices = idx2d.reshape(B)
    quantized = _make_sc_gather()(codebook, encoding_indices)
    mse = loss_sum[0, 0] / jnp.float32(B * D)
    loss = mse + jnp.float32(0.9) * mse
    return (quantized, loss, encoding_indices)


# speedup vs baseline: 1.1384x; 1.1384x over previous
import functools

import jax
import jax.numpy as jnp
from jax import lax
from jax.experimental import pallas as pl
from jax.experimental.pallas import tpu as pltpu
from jax.experimental.pallas import tpu_sc as plsc

B = 131072
D = 64
K = 512

BLOCK_B = 4096
GRID = B // BLOCK_B

NUM_WORKERS = 32
BPW = B // NUM_WORKERS
CHUNK = 1024
NCHUNK = BPW // CHUNK


def _argmin_body(x_ref, cbt2_ref, cnorm_ref, idx_ref, loss_ref, acc_ref):
    i = pl.program_id(0)
    x = x_ref[...]
    dot2 = lax.dot_general(x, cbt2_ref[...], (((1,), (0,)), ((), ())),
                           preferred_element_type=jnp.float32)
    dist = dot2 + cnorm_ref[...]
    minval = jnp.min(dist, axis=1, keepdims=True)
    iota = lax.broadcasted_iota(jnp.int32, dist.shape, 1).astype(jnp.float32)
    idxf = jnp.min(jnp.where(dist == minval, iota, jnp.float32(K)),
                   axis=1, keepdims=True)
    idx_ref[...] = idxf.astype(jnp.int32)
    xnorm = jnp.sum(x * x, axis=1, keepdims=True)

    @pl.when(i == 0)
    def _():
        acc_ref[...] = jnp.zeros_like(acc_ref)

    acc_ref[...] += minval + xnorm

    @pl.when(i == GRID - 1)
    def _():
        loss_ref[...] = jnp.sum(acc_ref[...], keepdims=True)


def _tc_argmin(inputs, cbt2, cnorm):
    return pl.pallas_call(
        _argmin_body,
        grid=(GRID,),
        in_specs=[
            pl.BlockSpec((BLOCK_B, D), lambda i: (i, 0)),
            pl.BlockSpec((D, K), lambda i: (0, 0)),
            pl.BlockSpec((1, K), lambda i: (0, 0)),
        ],
        out_specs=[
            pl.BlockSpec((BLOCK_B, 1), lambda i: (i, 0)),
            pl.BlockSpec((1, 1), lambda i: (0, 0)),
        ],
        out_shape=[
            jax.ShapeDtypeStruct((B, 1), jnp.int32),
            jax.ShapeDtypeStruct((1, 1), jnp.float32),
        ],
        scratch_shapes=[pltpu.VMEM((BLOCK_B, 1), jnp.float32)],
    )(inputs, cbt2, cnorm)


def _sc_gather_body(table_hbm, idx_hbm, out_hbm, idx_v, rows_v, sem):
    info = plsc.get_sparse_core_info()
    wid = lax.axis_index("s") * info.num_cores + lax.axis_index("c")
    base = wid * BPW
    pltpu.sync_copy(idx_hbm.at[pl.ds(base, BPW)], idx_v)
    for c in range(NCHUNK):
        pltpu.async_copy(
            table_hbm.at[idx_v.at[pl.ds(c * CHUNK, CHUNK)]], rows_v, sem
        ).wait()
        pltpu.sync_copy(rows_v, out_hbm.at[pl.ds(base + c * CHUNK, CHUNK)])


@functools.lru_cache(maxsize=1)
def _make_sc_gather():
    return pl.kernel(
        _sc_gather_body,
        mesh=plsc.VectorSubcoreMesh(core_axis_name="c", subcore_axis_name="s"),
        compiler_params=pltpu.CompilerParams(use_tc_tiling_on_sc=False),
        out_type=jax.ShapeDtypeStruct((B, D), jnp.float32),
        scratch_types=[
            pltpu.VMEM((BPW,), jnp.int32),
            pltpu.VMEM((CHUNK, D), jnp.float32),
            pltpu.SemaphoreType.DMA,
        ],
    )


def kernel(inputs, codebook):
    cbt2 = -2.0 * codebook.T
    cnorm = jnp.sum(codebook * codebook, axis=1)[None, :]
    idx2d, loss_sum = _tc_argmin(inputs, cbt2, cnorm)
    encoding_indices = idx2d.reshape(B)
    quantized = _make_sc_gather()(codebook, encoding_indices)
    mse = loss_sum[0, 0] / jnp.float32(B * D)
    loss = mse + jnp.float32(0.9) * mse
    return (quantized, loss, encoding_indices)


# speedup vs baseline: 1.5550x; 1.3660x over previous
import functools

import jax
import jax.numpy as jnp
from jax import lax
from jax.experimental import pallas as pl
from jax.experimental.pallas import tpu as pltpu
from jax.experimental.pallas import tpu_sc as plsc

B = 131072
D = 64
K = 512

BLOCK_B = 4096
GRID = B // BLOCK_B

NUM_WORKERS = 32
BPW = B // NUM_WORKERS
CHUNK = 1024
NCHUNK = BPW // CHUNK


def _argmin_body(xt_ref, cb2_ref, cnorm_ref, idx_ref, loss_ref, acc_ref):
    i = pl.program_id(0)
    xt = xt_ref[...]
    dot2 = lax.dot_general(cb2_ref[...], xt, (((1,), (0,)), ((), ())),
                           preferred_element_type=jnp.float32)
    dist = dot2 + cnorm_ref[...]
    minval = jnp.min(dist, axis=0, keepdims=True)
    iota = lax.broadcasted_iota(jnp.int32, dist.shape, 0).astype(jnp.float32)
    idxf = jnp.min(jnp.where(dist == minval, iota, jnp.float32(K)),
                   axis=0, keepdims=True)
    idx_ref[...] = idxf.astype(jnp.int32)[None]
    xnorm = jnp.sum(xt * xt, axis=0, keepdims=True)

    @pl.when(i == 0)
    def _():
        acc_ref[...] = jnp.zeros_like(acc_ref)

    acc_ref[...] += minval + xnorm

    @pl.when(i == GRID - 1)
    def _():
        loss_ref[...] = jnp.sum(acc_ref[...], keepdims=True)


def _tc_argmin(xt, cb2, cnorm):
    return pl.pallas_call(
        _argmin_body,
        grid=(GRID,),
        in_specs=[
            pl.BlockSpec((D, BLOCK_B), lambda i: (0, i)),
            pl.BlockSpec((K, D), lambda i: (0, 0)),
            pl.BlockSpec((K, 1), lambda i: (0, 0)),
        ],
        out_specs=[
            pl.BlockSpec((1, 1, BLOCK_B), lambda i: (i, 0, 0)),
            pl.BlockSpec((1, 1), lambda i: (0, 0)),
        ],
        out_shape=[
            jax.ShapeDtypeStruct((GRID, 1, BLOCK_B), jnp.int32),
            jax.ShapeDtypeStruct((1, 1), jnp.float32),
        ],
        scratch_shapes=[pltpu.VMEM((1, BLOCK_B), jnp.float32)],
    )(xt, cb2, cnorm)


def _sc_gather_body(table_hbm, idx_hbm, out_hbm, idx_v, rows_v, sem):
    info = plsc.get_sparse_core_info()
    wid = lax.axis_index("s") * info.num_cores + lax.axis_index("c")
    base = wid * BPW
    pltpu.sync_copy(idx_hbm.at[pl.ds(base, BPW)], idx_v)
    for c in range(NCHUNK):
        pltpu.async_copy(
            table_hbm.at[idx_v.at[pl.ds(c * CHUNK, CHUNK)]], rows_v, sem
        ).wait()
        pltpu.sync_copy(rows_v, out_hbm.at[pl.ds(base + c * CHUNK, CHUNK)])


@functools.lru_cache(maxsize=1)
def _make_sc_gather():
    return pl.kernel(
        _sc_gather_body,
        mesh=plsc.VectorSubcoreMesh(core_axis_name="c", subcore_axis_name="s"),
        compiler_params=pltpu.CompilerParams(use_tc_tiling_on_sc=False),
        out_type=jax.ShapeDtypeStruct((B, D), jnp.float32),
        scratch_types=[
            pltpu.VMEM((BPW,), jnp.int32),
            pltpu.VMEM((CHUNK, D), jnp.float32),
            pltpu.SemaphoreType.DMA,
        ],
    )


def kernel(inputs, codebook):
    cb2 = -2.0 * codebook
    cnorm = jnp.sum(codebook * codebook, axis=1)[:, None]
    idx3d, loss_sum = _tc_argmin(inputs.T, cb2, cnorm)
    encoding_indices = idx3d.reshape(B)
    quantized = _make_sc_gather()(codebook, encoding_indices)
    mse = loss_sum[0, 0] / jnp.float32(B * D)
    loss = mse + jnp.float32(0.9) * mse
    return (quantized, loss, encoding_indices)


# speedup vs baseline: 1.8428x; 1.1851x over previous
import functools

import jax
import jax.numpy as jnp
from jax import lax
from jax.experimental import pallas as pl
from jax.experimental.pallas import tpu as pltpu
from jax.experimental.pallas import tpu_sc as plsc

B = 131072
D = 64
K = 512

BLOCK_B = 4096
GRID = B // BLOCK_B

NUM_WORKERS = 32
BPW = B // NUM_WORKERS
CCOL = 512
NCHUNK = BPW // CCOL


def _argmin_body(xt_ref, cb2_ref, cnorm_ref, idx_ref, loss_ref, acc_ref):
    i = pl.program_id(0)
    xt = xt_ref[...]
    dot2 = lax.dot_general(cb2_ref[...], xt, (((1,), (0,)), ((), ())),
                           preferred_element_type=jnp.float32)
    dist = dot2 + cnorm_ref[...]
    minval = jnp.min(dist, axis=0, keepdims=True)
    iota = lax.broadcasted_iota(jnp.int32, dist.shape, 0).astype(jnp.float32)
    idxf = jnp.min(jnp.where(dist == minval, iota, jnp.float32(K)),
                   axis=0, keepdims=True)
    idx_ref[...] = idxf.astype(jnp.int32)[None]
    xnorm = jnp.sum(xt * xt, axis=0, keepdims=True)

    @pl.when(i == 0)
    def _():
        acc_ref[...] = jnp.zeros_like(acc_ref)

    acc_ref[...] += minval + xnorm

    @pl.when(i == GRID - 1)
    def _():
        loss_ref[...] = jnp.sum(acc_ref[...], keepdims=True)


def _tc_argmin(xt, cb2, cnorm):
    return pl.pallas_call(
        _argmin_body,
        grid=(GRID,),
        in_specs=[
            pl.BlockSpec((D, BLOCK_B), lambda i: (0, i)),
            pl.BlockSpec((K, D), lambda i: (0, 0)),
            pl.BlockSpec((K, 1), lambda i: (0, 0)),
        ],
        out_specs=[
            pl.BlockSpec((1, 1, BLOCK_B), lambda i: (i, 0, 0)),
            pl.BlockSpec((1, 1), lambda i: (0, 0)),
        ],
        out_shape=[
            jax.ShapeDtypeStruct((GRID, 1, BLOCK_B), jnp.int32),
            jax.ShapeDtypeStruct((1, 1), jnp.float32),
        ],
        scratch_shapes=[pltpu.VMEM((1, BLOCK_B), jnp.float32)],
    )(xt, cb2, cnorm)


def _sc_gather_body(tt_hbm, idx_hbm, out_hbm, idx_v, tt_v, lt0, lt1, sem0, sem1):
    info = plsc.get_sparse_core_info()
    wid = lax.axis_index("s") * info.num_cores + lax.axis_index("c")
    base = wid * BPW
    pltpu.sync_copy(idx_hbm.at[pl.ds(base, BPW)], idx_v)
    pltpu.sync_copy(tt_hbm, tt_v)
    lts = (lt0, lt1)
    sems = (sem0, sem1)
    pending = [None, None]
    dvecs = [jnp.full((16,), d, jnp.int32) for d in range(D)]

    for c in range(NCHUNK):
        bsel = c % 2
        if pending[bsel] is not None:
            pending[bsel].wait()
        lt = lts[bsel]

        def jbody(j, _, lt=lt, c=c):
            i16 = idx_v[pl.ds(c * CCOL + j * 16, 16)]
            for d in range(D):
                lt[d, pl.ds(j * 16, 16)] = plsc.load_gather(
                    tt_v, [dvecs[d], i16])
            return 0

        lax.fori_loop(0, CCOL // 16, jbody, 0)
        cp = pltpu.make_async_copy(
            lt, out_hbm.at[:, pl.ds(base + c * CCOL, CCOL)], sems[bsel])
        cp.start()
        pending[bsel] = cp
    for cp in pending:
        if cp is not None:
            cp.wait()


@functools.lru_cache(maxsize=1)
def _make_sc_gather():
    return pl.kernel(
        _sc_gather_body,
        mesh=plsc.VectorSubcoreMesh(core_axis_name="c", subcore_axis_name="s"),
        compiler_params=pltpu.CompilerParams(
            use_tc_tiling_on_sc=False, needs_layout_passes=False),
        out_type=jax.ShapeDtypeStruct((D, B), jnp.float32),
        scratch_types=[
            pltpu.VMEM((BPW,), jnp.int32),
            pltpu.VMEM((D, K), jnp.float32),
            pltpu.VMEM((D, CCOL), jnp.float32),
            pltpu.VMEM((D, CCOL), jnp.float32),
            pltpu.SemaphoreType.DMA,
            pltpu.SemaphoreType.DMA,
        ],
    )


def kernel(inputs, codebook):
    cb2 = -2.0 * codebook
    cnorm = jnp.sum(codebook * codebook, axis=1)[:, None]
    idx3d, loss_sum = _tc_argmin(inputs.T, cb2, cnorm)
    encoding_indices = idx3d.reshape(B)
    qt = _make_sc_gather()(codebook.T, encoding_indices)
    quantized = qt.T
    mse = loss_sum[0, 0] / jnp.float32(B * D)
    loss = mse + jnp.float32(0.9) * mse
    return (quantized, loss, encoding_indices)
